# baseline (speedup 1.0000x reference)
import functools

import jax
import jax.numpy as jnp
from jax import lax
from jax.experimental import pallas as pl
from jax.experimental.pallas import tpu as pltpu
from jax.experimental.pallas import tpu_sc as plsc

N = 10000
D = 128
E = 320000

NC = 2
NS = 16
NW = NC * NS
K = 128
GS = 4
KQ = K // GS
CHUNKS = 80
TOTCH = NW * CHUNKS
CH0 = 40
CH1 = 160 - CH0
CHMAX = max(CH0, CH1)
E_PAD = TOTCH * K
TOTCH_PAD = TOTCH + CHMAX
QD = 8
N_ACC = 10240
ZROWS = N_ACC // NS


@functools.cache
def _sc_kernels():
    mesh = plsc.VectorSubcoreMesh(core_axis_name="c", subcore_axis_name="s",
                                  num_cores=NC, num_subcores=NS)

    @functools.partial(
        pl.kernel,
        out_type=jax.ShapeDtypeStruct((NC, N_ACC, D), jnp.float32),
        mesh=mesh,
        scratch_types=[
            pltpu.VMEM_SHARED((N_ACC, D), jnp.float32),
            pltpu.VMEM((CHMAX, K), jnp.int32),
            pltpu.VMEM((2, K), jnp.int32),
            pltpu.VMEM((2, K, D), jnp.float32),
            pltpu.SemaphoreType.DMA,
            pltpu.SemaphoreType.DMA,
            pltpu.SemaphoreType.DMA,
        ],
    )
    def _agg(x_hbm, srcp, dstp, z128, out_p,
             acc, dst_all, si, rows, isem, gsem, ssem):
        c = lax.axis_index("c")
        s = lax.axis_index("s")
        nch = jnp.where(c == 0, CH0, CH1)
        base = jnp.where(c == 0, s * CH0, NS * CH0 + s * CH1)
        pltpu.sync_copy(z128, acc.at[pl.ds(s * ZROWS, ZROWS)])
        pltpu.sync_copy(dstp.at[pl.ds(base, CHMAX)], dst_all)
        pltpu.sync_copy(srcp.at[base], si.at[0])
        plsc.subcore_barrier()

        def load_idx(m, j):
            pltpu.async_copy(srcp.at[base + m], si.at[j], isem)

        def wait_idx():
            pltpu.make_async_copy(srcp.at[base], si.at[0], isem).wait()

        def start_gather(j):
            for q in range(GS):
                pltpu.async_copy(x_hbm.at[si.at[j, pl.ds(q * KQ, KQ)]],
                                 rows.at[j, pl.ds(q * KQ, KQ)], gsem)

        def wait_gather():
            for _ in range(GS):
                pltpu.make_async_copy(x_hbm.at[si.at[0, pl.ds(0, KQ)]],
                                      rows.at[0, pl.ds(0, KQ)], gsem).wait()

        def start_scatter(m, j):
            pltpu.async_copy(rows.at[j], acc.at[dst_all.at[m]], ssem, add=True)

        def wait_scatter():
            pltpu.make_async_copy(rows.at[0], acc.at[dst_all.at[0]], ssem).wait()

        start_gather(0)
        load_idx(1, 1)

        def outer(g, carry):
            for jj in range(2):
                m = g * 2 + jj
                wait_gather()
                start_scatter(m, jj)

                @pl.when(m + 2 < nch)
                def _():
                    load_idx(m + 2, jj)

                @pl.when(m >= 1)
                def _():
                    wait_scatter()

                @pl.when(m + 1 < nch)
                def _():
                    wait_idx()
                    start_gather(1 - jj)
            return carry

        lax.fori_loop(0, nch // 2, outer, 0)
        wait_scatter()
        plsc.subcore_barrier()
        orow = s * ZROWS
        pltpu.sync_copy(acc.at[pl.ds(orow, ZROWS)], out_p.at[c, pl.ds(orow, ZROWS)])

    @functools.partial(
        pl.kernel,
        out_type=jax.ShapeDtypeStruct((NC, N_ACC, D), jnp.float32),
        mesh=mesh,
        scratch_types=[
            pltpu.VMEM_SHARED((N_ACC, D), jnp.float32),
            pltpu.VMEM((CHUNKS, K), jnp.int32),
            pltpu.VMEM((K, D), jnp.float32),
            pltpu.SemaphoreType.DMA,
        ],
    )
    def _cnt(dstp, z128, ones128, out_c, acc, dst_all, ones_v, ssem):
        c = lax.axis_index("c")
        s = lax.axis_index("s")
        wid = c * NS + s
        pltpu.sync_copy(z128, acc.at[pl.ds(s * ZROWS, ZROWS)])
        pltpu.sync_copy(ones128, ones_v)
        pltpu.sync_copy(dstp.at[pl.ds(wid * CHUNKS, CHUNKS)], dst_all)
        plsc.subcore_barrier()

        def wait_scatter():
            pltpu.make_async_copy(ones_v, acc.at[dst_all.at[0]], ssem).wait()

        def chunk(m, carry):
            pltpu.async_copy(ones_v, acc.at[dst_all.at[m]], ssem, add=True)

            @pl.when(m >= QD)
            def _():
                wait_scatter()
            return carry

        lax.fori_loop(0, CHUNKS, chunk, 0)
        for _ in range(QD):
            wait_scatter()
        plsc.subcore_barrier()
        orow = s * ZROWS
        pltpu.sync_copy(acc.at[pl.ds(orow, ZROWS)], out_c.at[c, pl.ds(orow, ZROWS)])

    return _agg, _cnt


RB = 1000
G = N // RB


def _combine_body(p_ref, c_ref, x_ref, wl_ref, wr_ref, b_ref, o_ref, *, relu):
    cnt = c_ref[0, :, 0:1] + c_ref[1, :, 0:1]
    inv = 1.0 / jnp.maximum(cnt, 1.0)
    mean = (p_ref[0] + p_ref[1]) * inv
    dn = (((1,), (1,)), ((), ()))
    acc = lax.dot_general(mean, wl_ref[...], dn, preferred_element_type=jnp.float32)
    acc = acc + lax.dot_general(x_ref[...], wr_ref[...], dn,
                                preferred_element_type=jnp.float32)
    acc = acc + b_ref[...]
    if relu:
        acc = jnp.maximum(acc, 0.0)
    o_ref[...] = acc


def _combine(p, cnt2, x, Wl, Wr, b, relu):
    return pl.pallas_call(
        functools.partial(_combine_body, relu=relu),
        grid=(G,),
        in_specs=[
            pl.BlockSpec((2, RB, D), lambda i: (0, i, 0)),
            pl.BlockSpec((2, RB, D), lambda i: (0, i, 0)),
            pl.BlockSpec((RB, D), lambda i: (i, 0)),
            pl.BlockSpec((D, D), lambda i: (0, 0)),
            pl.BlockSpec((D, D), lambda i: (0, 0)),
            pl.BlockSpec((1, D), lambda i: (0, 0)),
        ],
        out_specs=pl.BlockSpec((RB, D), lambda i: (i, 0)),
        out_shape=jax.ShapeDtypeStruct((N, D), jnp.float32),
    )(p, cnt2, x, Wl, Wr, b.reshape(1, D))


def kernel(x, edge_index, W1l, W1r, b1, W2l, W2r, b2, W3l, W3r, b3):
    src = edge_index[0]
    dst = edge_index[1]
    pad = TOTCH_PAD * K - E
    srcp = jnp.concatenate([src, jnp.zeros((pad,), jnp.int32)]).reshape(TOTCH_PAD, K)
    dstp = jnp.concatenate([dst, jnp.full((pad,), N, jnp.int32)]).reshape(TOTCH_PAD, K)
    z128 = jnp.zeros((ZROWS, D), jnp.float32)
    ones128 = jnp.ones((K, D), jnp.float32)

    agg_fn, cnt_fn = _sc_kernels()
    cnt2 = cnt_fn(dstp, z128, ones128)
    p1 = agg_fn(x, srcp, dstp, z128)
    h1 = _combine(p1, cnt2, x, W1l, W1r, b1, relu=True)
    p2 = agg_fn(h1, srcp, dstp, z128)
    h2 = _combine(p2, cnt2, h1, W2l, W2r, b2, relu=True)
    p3 = agg_fn(h2, srcp, dstp, z128)
    return _combine(p3, cnt2, h2, W3l, W3r, b3, relu=False)


# speedup vs baseline: 1.1753x; 1.1753x over previous
import functools

import jax
import jax.numpy as jnp
from jax import lax
from jax.experimental import pallas as pl
from jax.experimental.pallas import tpu as pltpu
from jax.experimental.pallas import tpu_sc as plsc

N = 10000
D = 128
E = 320000

NC = 2
NS = 16
NW = NC * NS
K = 128
GS = 4
KQ = K // GS
CHUNKS = 80
TOTCH = NW * CHUNKS
CH0 = 120
CH1 = 160 - CH0
CHMAX = max(CH0, CH1)
E_PAD = TOTCH * K
TOTCH_PAD = TOTCH + CHMAX
QD = 8
N_ACC = 10240
ZROWS = N_ACC // NS


@functools.cache
def _sc_kernels():
    mesh = plsc.VectorSubcoreMesh(core_axis_name="c", subcore_axis_name="s",
                                  num_cores=NC, num_subcores=NS)

    @functools.partial(
        pl.kernel,
        out_type=jax.ShapeDtypeStruct((NC, N_ACC, D), jnp.float32),
        mesh=mesh,
        scratch_types=[
            pltpu.VMEM_SHARED((N_ACC, D), jnp.float32),
            pltpu.VMEM((CHMAX, K), jnp.int32),
            pltpu.VMEM((2, K), jnp.int32),
            pltpu.VMEM((2, K, D), jnp.float32),
            pltpu.SemaphoreType.DMA,
            pltpu.SemaphoreType.DMA,
            pltpu.SemaphoreType.DMA,
        ],
    )
    def _agg(x_hbm, srcp, dstp, z128, out_p,
             acc, dst_all, si, rows, isem, gsem, ssem):
        c = lax.axis_index("c")
        s = lax.axis_index("s")
        nch = jnp.where(c == 0, CH0, CH1)
        base = jnp.where(c == 0, s * CH0, NS * CH0 + s * CH1)
        pltpu.sync_copy(z128, acc.at[pl.ds(s * ZROWS, ZROWS)])
        pltpu.sync_copy(dstp.at[pl.ds(base, CHMAX)], dst_all)
        pltpu.sync_copy(srcp.at[base], si.at[0])
        plsc.subcore_barrier()

        def load_idx(m, j):
            pltpu.async_copy(srcp.at[base + m], si.at[j], isem)

        def wait_idx():
            pltpu.make_async_copy(srcp.at[base], si.at[0], isem).wait()

        def start_gather(j):
            for q in range(GS):
                pltpu.async_copy(x_hbm.at[si.at[j, pl.ds(q * KQ, KQ)]],
                                 rows.at[j, pl.ds(q * KQ, KQ)], gsem)

        def wait_gather():
            for _ in range(GS):
                pltpu.make_async_copy(x_hbm.at[si.at[0, pl.ds(0, KQ)]],
                                      rows.at[0, pl.ds(0, KQ)], gsem).wait()

        def start_scatter(m, j):
            pltpu.async_copy(rows.at[j], acc.at[dst_all.at[m]], ssem, add=True)

        def wait_scatter():
            pltpu.make_async_copy(rows.at[0], acc.at[dst_all.at[0]], ssem).wait()

        start_gather(0)
        load_idx(1, 1)

        def outer(g, carry):
            for jj in range(2):
                m = g * 2 + jj
                wait_gather()
                start_scatter(m, jj)

                @pl.when(m + 2 < nch)
                def _():
                    load_idx(m + 2, jj)

                @pl.when(m >= 1)
                def _():
                    wait_scatter()

                @pl.when(m + 1 < nch)
                def _():
                    wait_idx()
                    start_gather(1 - jj)
            return carry

        lax.fori_loop(0, nch // 2, outer, 0)
        wait_scatter()
        plsc.subcore_barrier()
        orow = s * ZROWS
        pltpu.sync_copy(acc.at[pl.ds(orow, ZROWS)], out_p.at[c, pl.ds(orow, ZROWS)])

    @functools.partial(
        pl.kernel,
        out_type=jax.ShapeDtypeStruct((NC, N_ACC, D), jnp.float32),
        mesh=mesh,
        scratch_types=[
            pltpu.VMEM_SHARED((N_ACC, D), jnp.float32),
            pltpu.VMEM((CHUNKS, K), jnp.int32),
            pltpu.VMEM((K, D), jnp.float32),
            pltpu.SemaphoreType.DMA,
        ],
    )
    def _cnt(dstp, z128, ones128, out_c, acc, dst_all, ones_v, ssem):
        c = lax.axis_index("c")
        s = lax.axis_index("s")
        wid = c * NS + s
        pltpu.sync_copy(z128, acc.at[pl.ds(s * ZROWS, ZROWS)])
        pltpu.sync_copy(ones128, ones_v)
        pltpu.sync_copy(dstp.at[pl.ds(wid * CHUNKS, CHUNKS)], dst_all)
        plsc.subcore_barrier()

        def wait_scatter():
            pltpu.make_async_copy(ones_v, acc.at[dst_all.at[0]], ssem).wait()

        def chunk(m, carry):
            pltpu.async_copy(ones_v, acc.at[dst_all.at[m]], ssem, add=True)

            @pl.when(m >= QD)
            def _():
                wait_scatter()
            return carry

        lax.fori_loop(0, CHUNKS, chunk, 0)
        for _ in range(QD):
            wait_scatter()
        plsc.subcore_barrier()
        orow = s * ZROWS
        pltpu.sync_copy(acc.at[pl.ds(orow, ZROWS)], out_c.at[c, pl.ds(orow, ZROWS)])

    return _agg, _cnt


RB = 1000
G = N // RB


def _combine_body(p_ref, c_ref, x_ref, wl_ref, wr_ref, b_ref, o_ref, *, relu):
    cnt = c_ref[0, :, 0:1] + c_ref[1, :, 0:1]
    inv = 1.0 / jnp.maximum(cnt, 1.0)
    mean = (p_ref[0] + p_ref[1]) * inv
    dn = (((1,), (1,)), ((), ()))
    acc = lax.dot_general(mean, wl_ref[...], dn, preferred_element_type=jnp.float32)
    acc = acc + lax.dot_general(x_ref[...], wr_ref[...], dn,
                                preferred_element_type=jnp.float32)
    acc = acc + b_ref[...]
    if relu:
        acc = jnp.maximum(acc, 0.0)
    o_ref[...] = acc


def _combine(p, cnt2, x, Wl, Wr, b, relu):
    return pl.pallas_call(
        functools.partial(_combine_body, relu=relu),
        grid=(G,),
        in_specs=[
            pl.BlockSpec((2, RB, D), lambda i: (0, i, 0)),
            pl.BlockSpec((2, RB, D), lambda i: (0, i, 0)),
            pl.BlockSpec((RB, D), lambda i: (i, 0)),
            pl.BlockSpec((D, D), lambda i: (0, 0)),
            pl.BlockSpec((D, D), lambda i: (0, 0)),
            pl.BlockSpec((1, D), lambda i: (0, 0)),
        ],
        out_specs=pl.BlockSpec((RB, D), lambda i: (i, 0)),
        out_shape=jax.ShapeDtypeStruct((N, D), jnp.float32),
    )(p, cnt2, x, Wl, Wr, b.reshape(1, D))


def kernel(x, edge_index, W1l, W1r, b1, W2l, W2r, b2, W3l, W3r, b3):
    src = edge_index[0]
    dst = edge_index[1]
    pad = TOTCH_PAD * K - E
    srcp = jnp.concatenate([src, jnp.zeros((pad,), jnp.int32)]).reshape(TOTCH_PAD, K)
    dstp = jnp.concatenate([dst, jnp.full((pad,), N, jnp.int32)]).reshape(TOTCH_PAD, K)
    z128 = jnp.zeros((ZROWS, D), jnp.float32)
    ones128 = jnp.ones((K, D), jnp.float32)

    agg_fn, cnt_fn = _sc_kernels()
    cnt2 = cnt_fn(dstp, z128, ones128)
    p1 = agg_fn(x, srcp, dstp, z128)
    h1 = _combine(p1, cnt2, x, W1l, W1r, b1, relu=True)
    p2 = agg_fn(h1, srcp, dstp, z128)
    h2 = _combine(p2, cnt2, h1, W2l, W2r, b2, relu=True)
    p3 = agg_fn(h2, srcp, dstp, z128)
    return _combine(p3, cnt2, h2, W3l, W3r, b3, relu=False)


# speedup vs baseline: 1.2321x; 1.0483x over previous
import functools

import jax
import jax.numpy as jnp
from jax import lax
from jax.experimental import pallas as pl
from jax.experimental.pallas import tpu as pltpu
from jax.experimental.pallas import tpu_sc as plsc

N = 10000
D = 128
E = 320000

NC = 2
NS = 16
NW = NC * NS
K = 128
GS = 4
KQ = K // GS
CHUNKS = 80
TOTCH = NW * CHUNKS
CH0 = 96
CH1 = 160 - CH0
CHMAX = max(CH0, CH1)
E_PAD = TOTCH * K
TOTCH_PAD = TOTCH + CHMAX
QD = 8
N_ACC = 10240
ZROWS = N_ACC // NS


@functools.cache
def _sc_kernels():
    mesh = plsc.VectorSubcoreMesh(core_axis_name="c", subcore_axis_name="s",
                                  num_cores=NC, num_subcores=NS)

    @functools.partial(
        pl.kernel,
        out_type=jax.ShapeDtypeStruct((NC, N_ACC, D), jnp.float32),
        mesh=mesh,
        scratch_types=[
            pltpu.VMEM_SHARED((N_ACC, D), jnp.float32),
            pltpu.VMEM((CHMAX, K), jnp.int32),
            pltpu.VMEM((2, K), jnp.int32),
            pltpu.VMEM((2, K, D), jnp.float32),
            pltpu.SemaphoreType.DMA,
            pltpu.SemaphoreType.DMA,
            pltpu.SemaphoreType.DMA,
        ],
    )
    def _agg(x_hbm, srcp, dstp, z128, out_p,
             acc, dst_all, si, rows, isem, gsem, ssem):
        c = lax.axis_index("c")
        s = lax.axis_index("s")
        nch = jnp.where(c == 0, CH0, CH1)
        base = jnp.where(c == 0, s * CH0, NS * CH0 + s * CH1)
        pltpu.sync_copy(z128, acc.at[pl.ds(s * ZROWS, ZROWS)])
        pltpu.sync_copy(dstp.at[pl.ds(base, CHMAX)], dst_all)
        pltpu.sync_copy(srcp.at[base], si.at[0])
        plsc.subcore_barrier()

        def load_idx(m, j):
            pltpu.async_copy(srcp.at[base + m], si.at[j], isem)

        def wait_idx():
            pltpu.make_async_copy(srcp.at[base], si.at[0], isem).wait()

        def start_gather(j):
            for q in range(GS):
                pltpu.async_copy(x_hbm.at[si.at[j, pl.ds(q * KQ, KQ)]],
                                 rows.at[j, pl.ds(q * KQ, KQ)], gsem)

        def wait_gather():
            for _ in range(GS):
                pltpu.make_async_copy(x_hbm.at[si.at[0, pl.ds(0, KQ)]],
                                      rows.at[0, pl.ds(0, KQ)], gsem).wait()

        def start_scatter(m, j):
            pltpu.async_copy(rows.at[j], acc.at[dst_all.at[m]], ssem, add=True)

        def wait_scatter():
            pltpu.make_async_copy(rows.at[0], acc.at[dst_all.at[0]], ssem).wait()

        start_gather(0)
        load_idx(1, 1)

        def outer(g, carry):
            for jj in range(2):
                m = g * 2 + jj
                wait_gather()
                start_scatter(m, jj)

                @pl.when(m + 2 < nch)
                def _():
                    load_idx(m + 2, jj)

                @pl.when(m >= 1)
                def _():
                    wait_scatter()

                @pl.when(m + 1 < nch)
                def _():
                    wait_idx()
                    start_gather(1 - jj)
            return carry

        lax.fori_loop(0, nch // 2, outer, 0)
        wait_scatter()
        plsc.subcore_barrier()
        orow = s * ZROWS
        pltpu.sync_copy(acc.at[pl.ds(orow, ZROWS)], out_p.at[c, pl.ds(orow, ZROWS)])

    @functools.partial(
        pl.kernel,
        out_type=jax.ShapeDtypeStruct((NC, N_ACC, D), jnp.float32),
        mesh=mesh,
        scratch_types=[
            pltpu.VMEM_SHARED((N_ACC, D), jnp.float32),
            pltpu.VMEM((CHUNKS, K), jnp.int32),
            pltpu.VMEM((K, D), jnp.float32),
            pltpu.SemaphoreType.DMA,
        ],
    )
    def _cnt(dstp, z128, ones128, out_c, acc, dst_all, ones_v, ssem):
        c = lax.axis_index("c")
        s = lax.axis_index("s")
        wid = c * NS + s
        pltpu.sync_copy(z128, acc.at[pl.ds(s * ZROWS, ZROWS)])
        pltpu.sync_copy(ones128, ones_v)
        pltpu.sync_copy(dstp.at[pl.ds(wid * CHUNKS, CHUNKS)], dst_all)
        plsc.subcore_barrier()

        def wait_scatter():
            pltpu.make_async_copy(ones_v, acc.at[dst_all.at[0]], ssem).wait()

        def chunk(m, carry):
            pltpu.async_copy(ones_v, acc.at[dst_all.at[m]], ssem, add=True)

            @pl.when(m >= QD)
            def _():
                wait_scatter()
            return carry

        lax.fori_loop(0, CHUNKS, chunk, 0)
        for _ in range(QD):
            wait_scatter()
        plsc.subcore_barrier()
        orow = s * ZROWS
        pltpu.sync_copy(acc.at[pl.ds(orow, ZROWS)], out_c.at[c, pl.ds(orow, ZROWS)])

    return _agg, _cnt


RB = 1000
G = N // RB


def _combine_body(p_ref, c_ref, x_ref, wl_ref, wr_ref, b_ref, o_ref, *, relu):
    cnt = c_ref[0, :, 0:1] + c_ref[1, :, 0:1]
    inv = 1.0 / jnp.maximum(cnt, 1.0)
    mean = (p_ref[0] + p_ref[1]) * inv
    dn = (((1,), (1,)), ((), ()))
    acc = lax.dot_general(mean, wl_ref[...], dn, preferred_element_type=jnp.float32)
    acc = acc + lax.dot_general(x_ref[...], wr_ref[...], dn,
                                preferred_element_type=jnp.float32)
    acc = acc + b_ref[...]
    if relu:
        acc = jnp.maximum(acc, 0.0)
    o_ref[...] = acc


def _combine(p, cnt2, x, Wl, Wr, b, relu):
    return pl.pallas_call(
        functools.partial(_combine_body, relu=relu),
        grid=(G,),
        in_specs=[
            pl.BlockSpec((2, RB, D), lambda i: (0, i, 0)),
            pl.BlockSpec((2, RB, D), lambda i: (0, i, 0)),
            pl.BlockSpec((RB, D), lambda i: (i, 0)),
            pl.BlockSpec((D, D), lambda i: (0, 0)),
            pl.BlockSpec((D, D), lambda i: (0, 0)),
            pl.BlockSpec((1, D), lambda i: (0, 0)),
        ],
        out_specs=pl.BlockSpec((RB, D), lambda i: (i, 0)),
        out_shape=jax.ShapeDtypeStruct((N, D), jnp.float32),
    )(p, cnt2, x, Wl, Wr, b.reshape(1, D))


def kernel(x, edge_index, W1l, W1r, b1, W2l, W2r, b2, W3l, W3r, b3):
    src = edge_index[0]
    dst = edge_index[1]
    pad = TOTCH_PAD * K - E
    srcp = jnp.concatenate([src, jnp.zeros((pad,), jnp.int32)]).reshape(TOTCH_PAD, K)
    dstp = jnp.concatenate([dst, jnp.full((pad,), N, jnp.int32)]).reshape(TOTCH_PAD, K)
    z128 = jnp.zeros((ZROWS, D), jnp.float32)
    ones128 = jnp.ones((K, D), jnp.float32)

    agg_fn, cnt_fn = _sc_kernels()
    cnt2 = cnt_fn(dstp, z128, ones128)
    p1 = agg_fn(x, srcp, dstp, z128)
    h1 = _combine(p1, cnt2, x, W1l, W1r, b1, relu=True)
    p2 = agg_fn(h1, srcp, dstp, z128)
    h2 = _combine(p2, cnt2, h1, W2l, W2r, b2, relu=True)
    p3 = agg_fn(h2, srcp, dstp, z128)
    return _combine(p3, cnt2, h2, W3l, W3r, b3, relu=False)
